# Initial kernel scaffold; baseline (speedup 1.0000x reference)
import functools

import jax
import jax.numpy as jnp
from jax import lax
from jax.experimental import pallas as pl
from jax.experimental.pallas import tpu as pltpu
from jax.experimental.pallas import tpu_sc as plsc

N = 10000
E = 320000
IN = 128
HID = 128
LAT = 64

NC = 2
NS = 16
NW = NC * NS
EPT = E // NW
CK = 80
NCHUNK = EPT // CK
ROWS_T = N // NS
SSEG = 1000

_SLOPE = 0.2


def _lrelu(v):
    return jnp.maximum(v, _SLOPE * v)


def _row_dot(a, h):
    return lax.dot_general(a, h, (((1,), (1,)), ((), ())),
                           preferred_element_type=jnp.float32)


def _tc1_body(x_ref, w_ref, as_ref, ad_ref, h_ref, scal_ref):
    h = jnp.dot(x_ref[...], w_ref[...], preferred_element_type=jnp.float32)
    h_ref[...] = h
    asr = _row_dot(as_ref[...], h)
    adr = _row_dot(ad_ref[...], h)
    asm = jnp.max(asr)
    scal_ref[...] = jnp.concatenate(
        [asr, adr, jnp.full((1, N), asm, jnp.float32),
         jnp.zeros((1, N), jnp.float32)], axis=0)


def _tc1(x, W1, a_s, a_d):
    return pl.pallas_call(
        _tc1_body,
        out_shape=(jax.ShapeDtypeStruct((N, HID), jnp.float32),
                   jax.ShapeDtypeStruct((4, N), jnp.float32)),
    )(x, W1, a_s, a_d)


def _tc2_body(outR_ref, outS_ref, b1_ref, wmu_ref, asmu_ref, admu_ref,
              wlv_ref, aslv_ref, adlv_ref, hcat_ref, scal_ref):
    acc = outR_ref[0] + outR_ref[1]
    s1 = jnp.reshape(outS_ref[0] + outS_ref[1], (N, 1))
    o = acc / (s1 + 1e-16) + b1_ref[...]
    hh = jnp.where(o > 0, o, jnp.exp(jnp.minimum(o, 0.0)) - 1.0)
    hmu = jnp.dot(hh, wmu_ref[...], preferred_element_type=jnp.float32)
    hlv = jnp.dot(hh, wlv_ref[...], preferred_element_type=jnp.float32)
    hcat_ref[...] = jnp.concatenate([hmu, hlv], axis=1)
    asmu = _row_dot(asmu_ref[...], hmu)
    admu = _row_dot(admu_ref[...], hmu)
    aslv = _row_dot(aslv_ref[...], hlv)
    adlv = _row_dot(adlv_ref[...], hlv)
    scal_ref[...] = jnp.concatenate(
        [asmu, admu, aslv, adlv,
         jnp.full((1, N), jnp.max(asmu), jnp.float32),
         jnp.full((1, N), jnp.max(aslv), jnp.float32),
         jnp.zeros((2, N), jnp.float32)], axis=0)


def _tc2(outR, outS, b1, W_mu, a_s_mu, a_d_mu, W_lv, a_s_lv, a_d_lv):
    return pl.pallas_call(
        _tc2_body,
        out_shape=(jax.ShapeDtypeStruct((N, 2 * LAT), jnp.float32),
                   jax.ShapeDtypeStruct((8, N), jnp.float32)),
    )(outR, outS, b1, W_mu, a_s_mu, a_d_mu, W_lv, a_s_lv, a_d_lv)


def _tc3_body(outR_ref, outSmu_ref, outSlv_ref, bmu_ref, blv_ref,
              mu_ref, lv_ref):
    acc = outR_ref[0] + outR_ref[1]
    smu = jnp.reshape(outSmu_ref[0] + outSmu_ref[1], (N, 1))
    slv = jnp.reshape(outSlv_ref[0] + outSlv_ref[1], (N, 1))
    mu_ref[...] = acc[:, :LAT] / (smu + 1e-16) + bmu_ref[...]
    lv_ref[...] = acc[:, LAT:] / (slv + 1e-16) + blv_ref[...]


def _tc3(outR, outSmu, outSlv, b_mu, b_lv):
    return pl.pallas_call(
        _tc3_body,
        out_shape=(jax.ShapeDtypeStruct((N, LAT), jnp.float32),
                   jax.ShapeDtypeStruct((N, LAT), jnp.float32)),
    )(outR, outSmu, outSlv, b_mu, b_lv)


def _make_sc_edge(nh):
    mesh = plsc.VectorSubcoreMesh(core_axis_name="c", subcore_axis_name="s",
                                  num_cores=NC, num_subcores=NS)
    out_type = [jax.ShapeDtypeStruct((NC, N, HID), jnp.float32)]
    out_type += [jax.ShapeDtypeStruct((NC, N), jnp.float32)] * nh
    scratch = []
    scratch += [pltpu.VMEM((N,), jnp.float32)] * (2 * nh)
    scratch += [pltpu.VMEM((16,), jnp.float32)] * nh
    scratch += [pltpu.VMEM((CK,), jnp.int32)] * 2
    scratch += [pltpu.VMEM((CK,), jnp.float32)] * nh
    scratch += [pltpu.VMEM((CK, HID), jnp.float32)]
    scratch += [pltpu.VMEM_SHARED((N, HID), jnp.float32)]
    scratch += [pltpu.VMEM_SHARED((N,), jnp.float32)] * nh
    scratch += [pltpu.SemaphoreType.DMA]

    def body(src_hbm, dst_hbm, scal_hbm, feat_hbm, zR_hbm, zS_hbm, *rest):
        outR, rest = rest[0], rest[1:]
        outS, rest = rest[:nh], rest[nh:]
        a_s = [rest[2 * i] for i in range(nh)]
        a_d = [rest[2 * i + 1] for i in range(nh)]
        rest = rest[2 * nh:]
        asmr, rest = rest[:nh], rest[nh:]
        srcv, dstv = rest[0], rest[1]
        rest = rest[2:]
        pref, rest = rest[:nh], rest[nh:]
        rows, accR = rest[0], rest[1]
        accS = rest[2:2 + nh]
        sem = rest[2 + nh]

        c = lax.axis_index("c")
        s = lax.axis_index("s")
        wid = s * NC + c

        pltpu.sync_copy(zR_hbm.at[pl.ds(s * ROWS_T, ROWS_T)],
                        accR.at[pl.ds(s * ROWS_T, ROWS_T)])

        @pl.when(s < N // SSEG)
        def _():
            for hh in range(nh):
                pltpu.sync_copy(zS_hbm.at[pl.ds(s * SSEG, SSEG)],
                                accS[hh].at[pl.ds(s * SSEG, SSEG)])

        for hh in range(nh):
            pltpu.sync_copy(scal_hbm.at[2 * hh], a_s[hh])
            pltpu.sync_copy(scal_hbm.at[2 * hh + 1], a_d[hh])
            pltpu.sync_copy(scal_hbm.at[2 * nh + hh, pl.ds(0, 16)], asmr[hh])
        plsc.subcore_barrier()

        asmv = [asmr[hh][...] for hh in range(nh)]
        ebase = wid * EPT

        def chunk(i, carry):
            off = ebase + i * CK
            pltpu.sync_copy(src_hbm.at[pl.ds(off, CK)], srcv)
            pltpu.sync_copy(dst_hbm.at[pl.ds(off, CK)], dstv)
            cp = pltpu.async_copy(feat_hbm.at[srcv], rows, sem)
            for j in range(CK // 16):
                s16 = srcv[pl.ds(16 * j, 16)]
                d16 = dstv[pl.ds(16 * j, 16)]
                for hh in range(nh):
                    av = plsc.load_gather(a_s[hh], [s16])
                    dv = plsc.load_gather(a_d[hh], [d16])
                    e = _lrelu(av + dv)
                    shift = _lrelu(dv + asmv[hh])
                    pref[hh][pl.ds(16 * j, 16)] = jnp.exp(e - shift)
            cp.wait()
            blocks = HID // 16 // nh
            for i2 in range(CK):
                for hh in range(nh):
                    pb = plsc.load_gather(
                        pref[hh], [jnp.full((16,), i2, jnp.int32)])
                    for v in range(blocks):
                        col = (hh * blocks + v) * 16
                        rows[i2, pl.ds(col, 16)] = rows[i2, pl.ds(col, 16)] * pb
            pltpu.sync_copy(rows, accR.at[dstv], add=True)
            for hh in range(nh):
                pltpu.sync_copy(pref[hh], accS[hh].at[dstv], add=True)
            return carry

        lax.fori_loop(0, NCHUNK, chunk, 0)
        plsc.subcore_barrier()

        pltpu.sync_copy(accR.at[pl.ds(s * ROWS_T, ROWS_T)],
                        outR.at[c, pl.ds(s * ROWS_T, ROWS_T)])

        @pl.when(s < N // SSEG)
        def _():
            for hh in range(nh):
                pltpu.sync_copy(accS[hh].at[pl.ds(s * SSEG, SSEG)],
                                outS[hh].at[c, pl.ds(s * SSEG, SSEG)])

    return pl.kernel(body, out_type=out_type, mesh=mesh,
                     scratch_types=scratch)


_sc_edge1 = _make_sc_edge(1)
_sc_edge2 = _make_sc_edge(2)


def kernel(x, edge_index, W1, a_src1, a_dst1, b1, W_mu, a_src_mu, a_dst_mu,
           b_mu, W_lv, a_src_lv, a_dst_lv, b_lv):
    src = edge_index[0]
    dst = edge_index[1]
    zR = jnp.zeros((N, HID), jnp.float32)
    zS = jnp.zeros((N,), jnp.float32)

    h, scal1 = _tc1(x, W1, a_src1, a_dst1)
    outR1, outS1 = _sc_edge1(src, dst, scal1, h, zR, zS)
    hcat, scal2 = _tc2(outR1, outS1, b1.reshape(1, HID), W_mu, a_src_mu,
                       a_dst_mu, W_lv, a_src_lv, a_dst_lv)
    outR2, outSmu, outSlv = _sc_edge2(src, dst, scal2, hcat, zR, zS)
    mu, lv = _tc3(outR2, outSmu, outSlv, b_mu.reshape(1, LAT),
                  b_lv.reshape(1, LAT))
    return (mu, lv)


# speedup vs baseline: 23.3842x; 23.3842x over previous
import jax
import jax.numpy as jnp
from jax import lax
from jax.experimental import pallas as pl
from jax.experimental.pallas import tpu as pltpu
from jax.experimental.pallas import tpu_sc as plsc

N = 10000
E = 320000
IN = 128
HID = 128
LAT = 64
HC = 64

NC = 2
NS = 16
EPT = E // NS
CK = 80
NCHUNK = EPT // CK
ROWS_T = 624
ROWS_TAIL = N - NS * ROWS_T
SSEG = 1000

_SLOPE = 0.2


def _lrelu(v):
    return jnp.maximum(v, _SLOPE * v)


def _row_dot(a, h):
    return lax.dot_general(a, h, (((1,), (1,)), ((), ())),
                           precision=lax.Precision.HIGHEST,
                           preferred_element_type=jnp.float32)


def _pack_scal(ref, rows):
    arr = jnp.concatenate(rows + [jnp.zeros((2, N), jnp.float32)], axis=0)
    ref[...] = arr.reshape(8, 1, N)


def _tc1_body(x_ref, w_ref, as_ref, ad_ref, h_ref, scal_ref):
    h = jnp.dot(x_ref[...], w_ref[...], precision=lax.Precision.HIGHEST,
                preferred_element_type=jnp.float32)
    h_ref[0] = h[:, :HC]
    h_ref[1] = h[:, HC:]
    asr = _row_dot(as_ref[...], h)
    adr = _row_dot(ad_ref[...], h)
    asm = jnp.full((1, N), jnp.max(asr), jnp.float32)
    _pack_scal(scal_ref, [asr, adr, asr, adr, asm, asm])


def _tc1(x, W1, a_s, a_d):
    return pl.pallas_call(
        _tc1_body,
        out_shape=(jax.ShapeDtypeStruct((NC, N, HC), jnp.float32),
                   jax.ShapeDtypeStruct((8, 1, N), jnp.float32)),
    )(x, W1, a_s, a_d)


def _tc2_body(outR_ref, outS_ref, b1_ref, wmu_ref, asmu_ref, admu_ref,
              wlv_ref, aslv_ref, adlv_ref, feat_ref, scal_ref):
    acc = jnp.concatenate([outR_ref[0], outR_ref[1]], axis=1)
    s1 = jnp.reshape(outS_ref[0], (N, 1))
    o = acc / (s1 + 1e-16) + b1_ref[...]
    hh = jnp.where(o > 0, o, jnp.exp(jnp.minimum(o, 0.0)) - 1.0)
    hmu = jnp.dot(hh, wmu_ref[...], precision=lax.Precision.HIGHEST,
                  preferred_element_type=jnp.float32)
    hlv = jnp.dot(hh, wlv_ref[...], precision=lax.Precision.HIGHEST,
                  preferred_element_type=jnp.float32)
    feat_ref[0] = hmu
    feat_ref[1] = hlv
    asmu = _row_dot(asmu_ref[...], hmu)
    admu = _row_dot(admu_ref[...], hmu)
    aslv = _row_dot(aslv_ref[...], hlv)
    adlv = _row_dot(adlv_ref[...], hlv)
    _pack_scal(scal_ref, [asmu, admu, aslv, adlv,
                          jnp.full((1, N), jnp.max(asmu), jnp.float32),
                          jnp.full((1, N), jnp.max(aslv), jnp.float32)])


def _tc2(outR, outS, b1, W_mu, a_s_mu, a_d_mu, W_lv, a_s_lv, a_d_lv):
    return pl.pallas_call(
        _tc2_body,
        out_shape=(jax.ShapeDtypeStruct((NC, N, HC), jnp.float32),
                   jax.ShapeDtypeStruct((8, 1, N), jnp.float32)),
    )(outR, outS, b1, W_mu, a_s_mu, a_d_mu, W_lv, a_s_lv, a_d_lv)


def _tc3_body(outR_ref, outS_ref, bmu_ref, blv_ref, mu_ref, lv_ref):
    smu = jnp.reshape(outS_ref[0], (N, 1))
    slv = jnp.reshape(outS_ref[1], (N, 1))
    mu_ref[...] = outR_ref[0] / (smu + 1e-16) + bmu_ref[...]
    lv_ref[...] = outR_ref[1] / (slv + 1e-16) + blv_ref[...]


def _tc3(outR, outS, b_mu, b_lv):
    return pl.pallas_call(
        _tc3_body,
        out_shape=(jax.ShapeDtypeStruct((N, LAT), jnp.float32),
                   jax.ShapeDtypeStruct((N, LAT), jnp.float32)),
    )(outR, outS, b_mu, b_lv)


def _sc_body(src_hbm, dst_hbm, scal_hbm, feat_hbm, zR_hbm, zS_hbm,
             outR, outS, asv, adv, asmr, srcv, dstv, pv, rows, accR,
             accS, sem):
    c = lax.axis_index("c")
    s = lax.axis_index("s")

    pltpu.sync_copy(zR_hbm, rows)
    for k in range(ROWS_T // CK):
        pltpu.sync_copy(rows, accR.at[pl.ds(s * ROWS_T + k * CK, CK)])
    rem = ROWS_T % CK
    if rem:
        pltpu.sync_copy(rows.at[pl.ds(0, rem)],
                        accR.at[pl.ds(s * ROWS_T + ROWS_T - rem, rem)])

    @pl.when(s == NS - 1)
    def _():
        pltpu.sync_copy(rows.at[pl.ds(0, ROWS_TAIL)],
                        accR.at[pl.ds(NS * ROWS_T, ROWS_TAIL)])

    @pl.when(s < N // SSEG)
    def _():
        pltpu.sync_copy(zS_hbm, pv)
        for k in range(SSEG // CK):
            pltpu.sync_copy(pv, accS.at[pl.ds(s * SSEG + k * CK, CK)])
        srem = SSEG % CK
        if srem:
            pltpu.sync_copy(pv.at[pl.ds(0, srem)],
                            accS.at[pl.ds(s * SSEG + SSEG - srem, srem)])

    pltpu.sync_copy(scal_hbm.at[2 * c, 0], asv)
    pltpu.sync_copy(scal_hbm.at[2 * c + 1, 0], adv)
    pltpu.sync_copy(scal_hbm.at[4 + c, 0, pl.ds(0, 16)], asmr)
    plsc.subcore_barrier()

    asmx = asmr[...]
    ebase = s * EPT

    def chunk(i, carry):
        off = ebase + i * CK
        pltpu.sync_copy(src_hbm.at[pl.ds(off, CK)], srcv)
        pltpu.sync_copy(dst_hbm.at[pl.ds(off, CK)], dstv)
        cp = pltpu.async_copy(feat_hbm.at[c].at[srcv], rows, sem)
        for j in range(CK // 16):
            s16 = srcv[pl.ds(16 * j, 16)]
            d16 = dstv[pl.ds(16 * j, 16)]
            av = plsc.load_gather(asv, [s16])
            dv = plsc.load_gather(adv, [d16])
            e = _lrelu(av + dv)
            shift = _lrelu(dv + asmx)
            pv[pl.ds(16 * j, 16)] = jnp.exp(e - shift)
        cp.wait()
        for j in range(CK // 16):
            p16 = pv[pl.ds(16 * j, 16)]
            for l in range(16):
                pb = jnp.take_along_axis(
                    p16, jnp.full((16,), l, jnp.int32), axis=0)
                i2 = 16 * j + l
                for v in range(HC // 16):
                    rows[i2, pl.ds(v * 16, 16)] = (
                        rows[i2, pl.ds(v * 16, 16)] * pb)
        pltpu.sync_copy(rows, accR.at[dstv], add=True)
        pltpu.sync_copy(pv, accS.at[dstv], add=True)
        return carry

    lax.fori_loop(0, NCHUNK, chunk, 0)
    plsc.subcore_barrier()

    def _rcopy(base, nrows):
        pltpu.sync_copy(accR.at[pl.ds(base, nrows)], rows.at[pl.ds(0, nrows)])
        pltpu.sync_copy(rows.at[pl.ds(0, nrows)],
                        outR.at[c, pl.ds(base, nrows)])

    for k in range(ROWS_T // CK):
        _rcopy(s * ROWS_T + k * CK, CK)
    if rem:
        _rcopy(s * ROWS_T + ROWS_T - rem, rem)

    @pl.when(s == NS - 1)
    def _():
        _rcopy(NS * ROWS_T, ROWS_TAIL)

    @pl.when(s < N // SSEG)
    def _():
        def _scopy(base, nel):
            pltpu.sync_copy(accS.at[pl.ds(base, nel)], pv.at[pl.ds(0, nel)])
            pltpu.sync_copy(pv.at[pl.ds(0, nel)],
                            outS.at[pl.ds(c * N + base, nel)])
        for k in range(SSEG // CK):
            _scopy(s * SSEG + k * CK, CK)
        srem = SSEG % CK
        if srem:
            _scopy(s * SSEG + SSEG - srem, srem)


_SC_OUT = (jax.ShapeDtypeStruct((NC, N, HC), jnp.float32),
           jax.ShapeDtypeStruct((NC * N,), jnp.float32))
_SC_SCRATCH = (
    pltpu.VMEM((N,), jnp.float32),
    pltpu.VMEM((N,), jnp.float32),
    pltpu.VMEM((16,), jnp.float32),
    pltpu.VMEM((CK,), jnp.int32),
    pltpu.VMEM((CK,), jnp.int32),
    pltpu.VMEM((CK,), jnp.float32),
    pltpu.VMEM((CK, HC), jnp.float32),
    pltpu.VMEM_SHARED((N, HC), jnp.float32),
    pltpu.VMEM_SHARED((N,), jnp.float32),
    pltpu.SemaphoreType.DMA,
)


def _sc_edge(src, dst, scal, feat, zR, zS):
    mesh = plsc.VectorSubcoreMesh(core_axis_name="c", subcore_axis_name="s",
                                  num_cores=NC, num_subcores=NS)
    return pl.kernel(
        _sc_body, out_type=_SC_OUT, mesh=mesh, scratch_types=_SC_SCRATCH,
        compiler_params=pltpu.CompilerParams(needs_layout_passes=False,
                                             use_tc_tiling_on_sc=False),
    )(src, dst, scal, feat, zR, zS)


def kernel(x, edge_index, W1, a_src1, a_dst1, b1, W_mu, a_src_mu, a_dst_mu,
           b_mu, W_lv, a_src_lv, a_dst_lv, b_lv):
    src = edge_index[0]
    dst = edge_index[1]
    zR = jnp.zeros((CK, HC), jnp.float32)
    zS = jnp.zeros((CK,), jnp.float32)

    h2, scal1 = _tc1(x, W1, a_src1, a_dst1)
    outR1, outS1 = _sc_edge(src, dst, scal1, h2, zR, zS)
    feat2, scal2 = _tc2(outR1, outS1.reshape(NC, N), b1.reshape(1, HID), W_mu, a_src_mu,
                        a_dst_mu, W_lv, a_src_lv, a_dst_lv)
    outR2, outS2 = _sc_edge(src, dst, scal2, feat2, zR, zS)
    mu, lv = _tc3(outR2, outS2.reshape(NC, N), b_mu.reshape(1, LAT),
                  b_lv.reshape(1, LAT))
    return (mu, lv)
